# baseline (speedup 1.0000x reference)
import functools

import jax
import jax.numpy as jnp
from jax import lax
from jax.experimental import pallas as pl
from jax.experimental.pallas import tpu as pltpu
from jax.experimental.pallas import tpu_sc as plsc

NUM_EMBEDDINGS = 8192
EMBED_DIM = 256
COMMITMENT_COST = 0.25

N_TOKENS = 16 * 1024
BT = 512
BK = 1024
G = N_TOKENS // BT

SEG_BOUNDS = (0, 2736, 5472, 8192)
NSEG = len(SEG_BOUNDS) - 1


def _argmin_body(x_ref, xsq_ref, w_ref, wsq_ref, idx_ref, dsum_ref):
    x = x_ref[...]
    xsq = xsq_ref[0, 0, :]
    seg_min = [jnp.full((BT,), jnp.inf, dtype=jnp.float32) for _ in range(NSEG)]
    seg_arg = [jnp.zeros((BT,), dtype=jnp.int32) for _ in range(NSEG)]
    for c in range(NUM_EMBEDDINGS // BK):
        w = w_ref[c * BK:(c + 1) * BK, :]
        wsq = wsq_ref[0, c * BK:(c + 1) * BK]
        s = lax.dot_general(x, w, (((1,), (1,)), ((), ())),
                            preferred_element_type=jnp.float32)
        dist = (xsq[:, None] + wsq[None, :]) - 2.0 * s
        col = lax.broadcasted_iota(jnp.int32, (BT, BK), 1)
        lo_j, hi_j = c * BK, (c + 1) * BK
        for sgi in range(NSEG):
            a, b = SEG_BOUNDS[sgi], SEG_BOUNDS[sgi + 1]
            if b <= lo_j or a >= hi_j:
                continue
            lo, hi = max(a - lo_j, 0), min(b - lo_j, BK)
            if lo == 0 and hi == BK:
                masked = dist
            else:
                inb = (col >= lo) & (col < hi)
                masked = jnp.where(inb, dist, jnp.inf)
            cmin = jnp.min(masked, axis=1)
            carg = jnp.argmin(masked, axis=1).astype(jnp.int32) + lo_j
            upd = cmin < seg_min[sgi]
            seg_min[sgi] = jnp.where(upd, cmin, seg_min[sgi])
            seg_arg[sgi] = jnp.where(upd, carg, seg_arg[sgi])
    acc = jnp.full((BT,), jnp.inf, dtype=jnp.float32)
    win = jnp.zeros((BT,), dtype=jnp.int32)
    winval = jnp.zeros((BT,), dtype=jnp.float32)
    for sgi in range(NSEG):
        m = seg_min[sgi] < acc
        rounded = seg_min[sgi].astype(jnp.bfloat16).astype(jnp.float32)
        acc = jnp.where(m, rounded, acc)
        win = jnp.where(m, seg_arg[sgi], win)
        winval = jnp.where(m, seg_min[sgi], winval)
    idx_ref[0, 0, :] = win
    dsum_ref[0, 0, :] = jnp.sum(winval.reshape(BT // 128, 128), axis=0)


def _argmin_call(flat, xsq, w, wsq):
    return pl.pallas_call(
        _argmin_body,
        grid=(G,),
        in_specs=[
            pl.BlockSpec((BT, EMBED_DIM), lambda i: (i, 0)),
            pl.BlockSpec((1, 1, BT), lambda i: (i, 0, 0)),
            pl.BlockSpec((NUM_EMBEDDINGS, EMBED_DIM), lambda i: (0, 0)),
            pl.BlockSpec((1, NUM_EMBEDDINGS), lambda i: (0, 0)),
        ],
        out_specs=[
            pl.BlockSpec((1, 1, BT), lambda i: (i, 0, 0)),
            pl.BlockSpec((1, 1, 128), lambda i: (i, 0, 0)),
        ],
        out_shape=[
            jax.ShapeDtypeStruct((G, 1, BT), jnp.int32),
            jax.ShapeDtypeStruct((G, 1, 128), jnp.float32),
        ],
        compiler_params=pltpu.CompilerParams(
            dimension_semantics=("parallel",)),
    )(flat, xsq, w, wsq)


def _sc_gather(table, idx):
    info = plsc.get_sparse_core_info()
    nw = info.num_cores * info.num_subcores
    b_per_w = N_TOKENS // nw
    chunk = 256
    mesh = plsc.VectorSubcoreMesh(core_axis_name="c", subcore_axis_name="s")

    @functools.partial(
        pl.kernel, mesh=mesh,
        out_type=jax.ShapeDtypeStruct((N_TOKENS, EMBED_DIM), jnp.float32),
        scratch_types=[
            pltpu.VMEM((b_per_w,), jnp.int32),
            pltpu.VMEM((chunk, EMBED_DIM), jnp.float32),
            pltpu.SemaphoreType.DMA,
        ],
    )
    def k(table_hbm, idx_hbm, out_hbm, idx_v, rows_v, sem):
        wid = lax.axis_index("s") * info.num_cores + lax.axis_index("c")
        base = wid * b_per_w
        pltpu.sync_copy(idx_hbm.at[pl.ds(base, b_per_w)], idx_v)
        for c in range(b_per_w // chunk):
            pltpu.async_copy(
                table_hbm.at[idx_v.at[pl.ds(c * chunk, chunk)]], rows_v, sem
            ).wait()
            pltpu.sync_copy(rows_v, out_hbm.at[pl.ds(base + c * chunk, chunk)])

    return k(table, idx)


def kernel(inputs, W):
    input_shape = inputs.shape
    flat_input = inputs.reshape(-1, 1, EMBED_DIM)
    flat = flat_input[:, 0, :]
    xsq = jnp.sum(flat ** 2, axis=1, keepdims=True)
    wsq = jnp.sum(W ** 2, axis=1)
    idx3, dsum = _argmin_call(
        flat, xsq.reshape(G, 1, BT), W, wsq.reshape(1, NUM_EMBEDDINGS))
    idx = idx3.reshape(N_TOKENS)
    quantized = _sc_gather(W, idx)
    loss = (COMMITMENT_COST / (N_TOKENS * EMBED_DIM)) * jnp.sum(dsum)
    return (loss, quantized.reshape(input_shape))
